# baseline (speedup 1.0000x reference)
import math

import jax
import jax.numpy as jnp
import numpy as np
from jax.experimental import pallas as pl
from jax.experimental.pallas import tpu as pltpu

_D = 768
_SEGS = ((0, 64), (64, 96), (96, 112))
_S_PAD = 128
_TILE = 2048


def _attn_kernel(q_ref, k_ref, v_ref, b_ref, m_ref, o_ref):
    q = q_ref[...].astype(jnp.bfloat16)
    k = k_ref[...]
    s = jax.lax.dot_general(
        q, k, (((1,), (1,)), ((), ())),
        preferred_element_type=jnp.float32,
    )
    s = s + b_ref[...]
    mx = jnp.max(s, axis=1, keepdims=True)
    e = jnp.exp(s - mx)
    denom = jax.lax.dot_general(
        e.astype(jnp.bfloat16), m_ref[...], (((1,), (0,)), ((), ())),
        preferred_element_type=jnp.float32,
    )
    p = (e / jnp.maximum(denom, 1e-30)).astype(jnp.bfloat16)
    o_ref[...] = jax.lax.dot_general(
        p, v_ref[...], (((1,), (0,)), ((), ())),
        preferred_element_type=jnp.float32,
    )


def kernel(query, keys0, values0, salience0, keys1, values1, salience1,
           keys2, values2, salience2):
    B, T, D = query.shape
    n = B * T
    q = query.reshape(n, D)
    k = jnp.concatenate([keys0, keys1, keys2], axis=0)
    v = jnp.concatenate([values0, values1, values2], axis=0)
    pad = _S_PAD - k.shape[0]
    k = (jnp.pad(k, ((0, pad), (0, 0))) * (1.0 / math.sqrt(_D))).astype(jnp.bfloat16)
    v = (jnp.pad(v, ((0, pad), (0, 0))) * (1.0 / 3.0)).astype(jnp.bfloat16)
    bias = jnp.pad(
        jnp.concatenate([salience0, salience1, salience2]), (0, pad),
        constant_values=-1e30,
    ).reshape(1, _S_PAD)
    seg_of = np.full((_S_PAD,), -1, dtype=np.int32)
    for si, (lo, hi) in enumerate(_SEGS):
        seg_of[lo:hi] = si
    seg_mat = jnp.asarray(
        (seg_of[:, None] == seg_of[None, :]) & (seg_of[:, None] >= 0),
        dtype=jnp.bfloat16)

    out = pl.pallas_call(
        _attn_kernel,
        grid=(n // _TILE,),
        in_specs=[
            pl.BlockSpec((_TILE, D), lambda i: (i, 0)),
            pl.BlockSpec((_S_PAD, D), lambda i: (0, 0)),
            pl.BlockSpec((_S_PAD, D), lambda i: (0, 0)),
            pl.BlockSpec((1, _S_PAD), lambda i: (0, 0)),
            pl.BlockSpec((_S_PAD, _S_PAD), lambda i: (0, 0)),
        ],
        out_specs=pl.BlockSpec((_TILE, D), lambda i: (i, 0)),
        out_shape=jax.ShapeDtypeStruct((n, D), jnp.float32),
        compiler_params=pltpu.CompilerParams(
            dimension_semantics=("parallel",)),
    )(q, k, v, bias, seg_mat)
    return out.reshape(B, T, D)


# speedup vs baseline: 1.0443x; 1.0443x over previous
import math

import jax
import jax.numpy as jnp
import numpy as np
from jax.experimental import pallas as pl
from jax.experimental.pallas import tpu as pltpu

_D = 768
_SEGS = ((0, 64), (64, 96), (96, 112))
_S_PAD = 128
_TILE = 2048
_SCALE = 1.0 / math.sqrt(_D)


def _attn_kernel(q_ref, k0_ref, k1_ref, k2_ref, v0_ref, v1_ref, v2_ref,
                 b_ref, m_ref, o_ref, kb_ref, vb_ref):
    @pl.when(pl.program_id(0) == 0)
    def _assemble():
        kb_ref[0:64, :] = (k0_ref[...] * _SCALE).astype(jnp.bfloat16)
        kb_ref[64:96, :] = (k1_ref[...] * _SCALE).astype(jnp.bfloat16)
        kb_ref[96:112, :] = (k2_ref[...] * _SCALE).astype(jnp.bfloat16)
        kb_ref[112:128, :] = jnp.zeros((16, _D), jnp.bfloat16)
        vb_ref[0:64, :] = (v0_ref[...] * (1.0 / 3.0)).astype(jnp.bfloat16)
        vb_ref[64:96, :] = (v1_ref[...] * (1.0 / 3.0)).astype(jnp.bfloat16)
        vb_ref[96:112, :] = (v2_ref[...] * (1.0 / 3.0)).astype(jnp.bfloat16)
        vb_ref[112:128, :] = jnp.zeros((16, _D), jnp.bfloat16)

    q = q_ref[...].astype(jnp.bfloat16)
    s = jax.lax.dot_general(
        q, kb_ref[...], (((1,), (1,)), ((), ())),
        preferred_element_type=jnp.float32,
    )
    s = s + b_ref[...]
    mx = jnp.max(s, axis=1, keepdims=True)
    e = jnp.exp(s - mx)
    denom = jax.lax.dot_general(
        e.astype(jnp.bfloat16), m_ref[...], (((1,), (0,)), ((), ())),
        preferred_element_type=jnp.float32,
    )
    p = (e / jnp.maximum(denom, 1e-30)).astype(jnp.bfloat16)
    o_ref[...] = jax.lax.dot_general(
        p, vb_ref[...], (((1,), (0,)), ((), ())),
        preferred_element_type=jnp.float32,
    )


def kernel(query, keys0, values0, salience0, keys1, values1, salience1,
           keys2, values2, salience2):
    B, T, D = query.shape
    n = B * T
    q = query.reshape(n, D)
    bias = jnp.pad(
        jnp.concatenate([salience0, salience1, salience2]),
        (0, _S_PAD - 112), constant_values=-1e30,
    ).reshape(1, _S_PAD)
    seg_of = np.full((_S_PAD,), -1, dtype=np.int32)
    for si, (lo, hi) in enumerate(_SEGS):
        seg_of[lo:hi] = si
    seg_mat = jnp.asarray(
        (seg_of[:, None] == seg_of[None, :]) & (seg_of[:, None] >= 0),
        dtype=jnp.bfloat16)

    whole = lambda i: (0, 0)
    out = pl.pallas_call(
        _attn_kernel,
        grid=(n // _TILE,),
        in_specs=[
            pl.BlockSpec((_TILE, D), lambda i: (i, 0)),
            pl.BlockSpec((64, D), whole),
            pl.BlockSpec((32, D), whole),
            pl.BlockSpec((16, D), whole),
            pl.BlockSpec((64, D), whole),
            pl.BlockSpec((32, D), whole),
            pl.BlockSpec((16, D), whole),
            pl.BlockSpec((1, _S_PAD), whole),
            pl.BlockSpec((_S_PAD, _S_PAD), whole),
        ],
        out_specs=pl.BlockSpec((_TILE, D), lambda i: (i, 0)),
        out_shape=jax.ShapeDtypeStruct((n, D), jnp.float32),
        scratch_shapes=[
            pltpu.VMEM((_S_PAD, _D), jnp.bfloat16),
            pltpu.VMEM((_S_PAD, _D), jnp.bfloat16),
        ],
    )(q, keys0, keys1, keys2, values0, values1, values2, bias, seg_mat)
    return out.reshape(B, T, D)


# speedup vs baseline: 1.0785x; 1.0328x over previous
import math

import jax
import jax.numpy as jnp
import numpy as np
from jax.experimental import pallas as pl
from jax.experimental.pallas import tpu as pltpu

_D = 768
_SEGS = ((0, 64), (64, 96), (96, 112))
_S_PAD = 128
_TILE = 2048
_SCALE = 1.0 / math.sqrt(_D)


def _attn_kernel(q_ref, k0_ref, k1_ref, k2_ref, v0_ref, v1_ref, v2_ref,
                 s0_ref, s1_ref, s2_ref, m_ref, o_ref, kb_ref, vb_ref,
                 b_ref):
    @pl.when(pl.program_id(0) == 0)
    def _assemble():
        kb_ref[0:64, :] = (k0_ref[...] * _SCALE).astype(jnp.bfloat16)
        kb_ref[64:96, :] = (k1_ref[...] * _SCALE).astype(jnp.bfloat16)
        kb_ref[96:112, :] = (k2_ref[...] * _SCALE).astype(jnp.bfloat16)
        kb_ref[112:128, :] = jnp.zeros((16, _D), jnp.bfloat16)
        vb_ref[0:64, :] = (v0_ref[...] * (1.0 / 3.0)).astype(jnp.bfloat16)
        vb_ref[64:96, :] = (v1_ref[...] * (1.0 / 3.0)).astype(jnp.bfloat16)
        vb_ref[96:112, :] = (v2_ref[...] * (1.0 / 3.0)).astype(jnp.bfloat16)
        vb_ref[112:128, :] = jnp.zeros((16, _D), jnp.bfloat16)
        b_ref[0:1, 0:64] = s0_ref[...]
        b_ref[0:1, 64:96] = s1_ref[...]
        b_ref[0:1, 96:112] = s2_ref[...]
        b_ref[0:1, 112:128] = jnp.full((1, 16), -1e30, jnp.float32)

    q = q_ref[...].astype(jnp.bfloat16)
    s = jax.lax.dot_general(
        q, kb_ref[...], (((1,), (1,)), ((), ())),
        preferred_element_type=jnp.float32,
    )
    s = s + b_ref[...]
    mx = jnp.max(s, axis=1, keepdims=True)
    e = jnp.exp(s - mx)
    denom = jax.lax.dot_general(
        e.astype(jnp.bfloat16), m_ref[...], (((1,), (0,)), ((), ())),
        preferred_element_type=jnp.float32,
    )
    p = (e / jnp.maximum(denom, 1e-30)).astype(jnp.bfloat16)
    o_ref[...] = jax.lax.dot_general(
        p, vb_ref[...], (((1,), (0,)), ((), ())),
        preferred_element_type=jnp.float32,
    )


def kernel(query, keys0, values0, salience0, keys1, values1, salience1,
           keys2, values2, salience2):
    B, T, D = query.shape
    n = B * T
    q = query.reshape(n, D)
    seg_of = np.full((_S_PAD,), -1, dtype=np.int32)
    for si, (lo, hi) in enumerate(_SEGS):
        seg_of[lo:hi] = si
    seg_mat = jnp.asarray(
        (seg_of[:, None] == seg_of[None, :]) & (seg_of[:, None] >= 0),
        dtype=jnp.bfloat16)

    whole = lambda i: (0, 0)
    out = pl.pallas_call(
        _attn_kernel,
        grid=(n // _TILE,),
        in_specs=[
            pl.BlockSpec((_TILE, D), lambda i: (i, 0)),
            pl.BlockSpec((64, D), whole),
            pl.BlockSpec((32, D), whole),
            pl.BlockSpec((16, D), whole),
            pl.BlockSpec((64, D), whole),
            pl.BlockSpec((32, D), whole),
            pl.BlockSpec((16, D), whole),
            pl.BlockSpec((1, 64), whole),
            pl.BlockSpec((1, 32), whole),
            pl.BlockSpec((1, 16), whole),
            pl.BlockSpec((_S_PAD, _S_PAD), whole),
        ],
        out_specs=pl.BlockSpec((_TILE, D), lambda i: (i, 0)),
        out_shape=jax.ShapeDtypeStruct((n, D), jnp.float32),
        scratch_shapes=[
            pltpu.VMEM((_S_PAD, _D), jnp.bfloat16),
            pltpu.VMEM((_S_PAD, _D), jnp.bfloat16),
            pltpu.VMEM((1, _S_PAD), jnp.float32),
        ],
    )(q, keys0, keys1, keys2, values0, values1, values2,
      salience0.reshape(1, 64), salience1.reshape(1, 32),
      salience2.reshape(1, 16), seg_mat)
    return out.reshape(B, T, D)
